# Initial kernel scaffold; baseline (speedup 1.0000x reference)
import jax
import jax.numpy as jnp
from jax.experimental import pallas as pl


def kernel(X, tables):
    raise NotImplementedError("write your pallas kernel here")


# speedup vs baseline: 1.0246x; 1.0246x over previous
import functools

import jax
import jax.numpy as jnp
from jax import lax
from jax.experimental import pallas as pl
from jax.experimental.pallas import tpu as pltpu
from jax.experimental.pallas import tpu_sc as plsc

NC = 2
NS = 16
NW = NC * NS
LANES = 16

CHUNK = 128


@functools.partial(jax.jit, static_argnames=("B", "F", "V", "H"))
def _embed_gather(X_flat, tables_flat, *, B, F, V, H):
    R = B * F
    rows_per_w = R // NW
    n_chunks = rows_per_w // CHUNK
    mesh = plsc.VectorSubcoreMesh(
        core_axis_name="c", subcore_axis_name="s",
        num_cores=NC, num_subcores=NS)

    @functools.partial(
        pl.kernel,
        out_type=jax.ShapeDtypeStruct((R, H), jnp.float32),
        mesh=mesh,
        scratch_types=[
            pltpu.VMEM((CHUNK,), jnp.int32),
            pltpu.VMEM((CHUNK,), jnp.int32),
            pltpu.VMEM((CHUNK, H), jnp.float32),
            pltpu.SemaphoreType.DMA,
        ],
        compiler_params=pltpu.CompilerParams(use_tc_tiling_on_sc=False),
    )
    def k(x_hbm, tab_hbm, out_hbm, xbuf, idx, rows, sem):
        wid = lax.axis_index("s") * NC + lax.axis_index("c")
        base = wid * rows_per_w

        def chunk_body(g, carry):
            r0 = base + g * CHUNK
            pltpu.sync_copy(x_hbm.at[pl.ds(r0, CHUNK)], xbuf)
            lane = lax.iota(jnp.int32, LANES)
            for j in range(CHUNK // LANES):
                r = r0 + j * LANES + lane
                fld = lax.rem(r, F)
                idx[pl.ds(j * LANES, LANES)] = (
                    xbuf[pl.ds(j * LANES, LANES)] + fld * V)
            pltpu.async_copy(tab_hbm.at[idx], rows, sem).wait()
            pltpu.sync_copy(rows, out_hbm.at[pl.ds(r0, CHUNK)])
            return carry

        lax.fori_loop(0, n_chunks, chunk_body, 0)

    return k(X_flat, tables_flat)


def kernel(X, tables):
    F, V, H = tables.shape
    B = X.shape[0]
    X_flat = X.reshape(B * F).astype(jnp.int32)
    tables_flat = tables.reshape(F * V, H)
    out = _embed_gather(X_flat, tables_flat, B=B, F=F, V=V, H=H)
    return out.reshape(B, F * H)


# speedup vs baseline: 1.0881x; 1.0619x over previous
import functools

import jax
import jax.numpy as jnp
from jax import lax
from jax.experimental import pallas as pl
from jax.experimental.pallas import tpu as pltpu
from jax.experimental.pallas import tpu_sc as plsc

NC = 2
NS = 16
NW = NC * NS
LANES = 16

CHUNK = 128
NBUF = 4


@functools.partial(jax.jit, static_argnames=("B", "F", "V", "H"))
def _embed_gather(X_flat, tables_flat, *, B, F, V, H):
    R = B * F
    rows_per_w = R // NW
    n_chunks = rows_per_w // CHUNK
    n_rounds = n_chunks // NBUF
    assert n_chunks % NBUF == 0
    mesh = plsc.VectorSubcoreMesh(
        core_axis_name="c", subcore_axis_name="s",
        num_cores=NC, num_subcores=NS)

    @functools.partial(
        pl.kernel,
        out_type=jax.ShapeDtypeStruct((R, H), jnp.float32),
        mesh=mesh,
        scratch_types=[
            pltpu.VMEM((NBUF, CHUNK), jnp.int32),
            pltpu.VMEM((NBUF, CHUNK), jnp.int32),
            pltpu.VMEM((NBUF, CHUNK, H), jnp.float32),
            [pltpu.SemaphoreType.DMA] * NBUF,
            [pltpu.SemaphoreType.DMA] * NBUF,
        ],
        compiler_params=pltpu.CompilerParams(use_tc_tiling_on_sc=False),
    )
    def k(x_hbm, tab_hbm, out_hbm, xbuf, idx, rows, gsems, osems):
        wid = lax.axis_index("s") * NC + lax.axis_index("c")
        base = wid * rows_per_w
        lane = lax.iota(jnp.int32, LANES)

        def round_body(p, carry):
            for s in range(NBUF):
                r0 = base + (p * NBUF + s) * CHUNK
                pltpu.sync_copy(x_hbm.at[pl.ds(r0, CHUNK)], xbuf.at[s])
                for j in range(CHUNK // LANES):
                    r = r0 + j * LANES + lane
                    fld = lax.rem(r, F)
                    idx[s, pl.ds(j * LANES, LANES)] = (
                        xbuf[s, pl.ds(j * LANES, LANES)] + fld * V)

                @pl.when(p > 0)
                def _drain_prev_write():
                    r_prev = base + ((p - 1) * NBUF + s) * CHUNK
                    pltpu.make_async_copy(
                        rows.at[s], out_hbm.at[pl.ds(r_prev, CHUNK)],
                        osems[s]).wait()

                pltpu.async_copy(tab_hbm.at[idx.at[s]], rows.at[s], gsems[s])
            for s in range(NBUF):
                r0 = base + (p * NBUF + s) * CHUNK
                pltpu.make_async_copy(
                    tab_hbm.at[idx.at[s]], rows.at[s], gsems[s]).wait()
                pltpu.async_copy(
                    rows.at[s], out_hbm.at[pl.ds(r0, CHUNK)], osems[s])
            return carry

        lax.fori_loop(0, n_rounds, round_body, 0)
        for s in range(NBUF):
            r_last = base + ((n_rounds - 1) * NBUF + s) * CHUNK
            pltpu.make_async_copy(
                rows.at[s], out_hbm.at[pl.ds(r_last, CHUNK)], osems[s]).wait()

    return k(X_flat, tables_flat)


def kernel(X, tables):
    F, V, H = tables.shape
    B = X.shape[0]
    X_flat = X.reshape(B * F).astype(jnp.int32)
    tables_flat = tables.reshape(F * V, H)
    out = _embed_gather(X_flat, tables_flat, B=B, F=F, V=V, H=H)
    return out.reshape(B, F * H)


# speedup vs baseline: 1.0906x; 1.0023x over previous
import functools

import jax
import jax.numpy as jnp
from jax import lax
from jax.experimental import pallas as pl
from jax.experimental.pallas import tpu as pltpu
from jax.experimental.pallas import tpu_sc as plsc

NC = 2
NS = 16
NW = NC * NS
LANES = 16

CHUNK = 128
NBUF = 8


@functools.partial(jax.jit, static_argnames=("B", "F", "V", "H"))
def _embed_gather(X_flat, tables_flat, *, B, F, V, H):
    R = B * F
    rows_per_w = R // NW
    n_chunks = rows_per_w // CHUNK
    n_rounds = n_chunks // NBUF
    assert n_chunks % NBUF == 0
    mesh = plsc.VectorSubcoreMesh(
        core_axis_name="c", subcore_axis_name="s",
        num_cores=NC, num_subcores=NS)

    @functools.partial(
        pl.kernel,
        out_type=jax.ShapeDtypeStruct((R, H), jnp.float32),
        mesh=mesh,
        scratch_types=[
            pltpu.VMEM((NBUF, CHUNK), jnp.int32),
            pltpu.VMEM((NBUF, CHUNK), jnp.int32),
            pltpu.VMEM((NBUF, CHUNK, H), jnp.float32),
            [pltpu.SemaphoreType.DMA] * NBUF,
            [pltpu.SemaphoreType.DMA] * NBUF,
        ],
        compiler_params=pltpu.CompilerParams(use_tc_tiling_on_sc=False),
    )
    def k(x_hbm, tab_hbm, out_hbm, xbuf, idx, rows, gsems, osems):
        wid = lax.axis_index("s") * NC + lax.axis_index("c")
        base = wid * rows_per_w
        lane = lax.iota(jnp.int32, LANES)

        def round_body(p, carry):
            for s in range(NBUF):
                r0 = base + (p * NBUF + s) * CHUNK
                pltpu.sync_copy(x_hbm.at[pl.ds(r0, CHUNK)], xbuf.at[s])
                for j in range(CHUNK // LANES):
                    r = r0 + j * LANES + lane
                    fld = lax.rem(r, F)
                    idx[s, pl.ds(j * LANES, LANES)] = (
                        xbuf[s, pl.ds(j * LANES, LANES)] + fld * V)

                @pl.when(p > 0)
                def _drain_prev_write():
                    r_prev = base + ((p - 1) * NBUF + s) * CHUNK
                    pltpu.make_async_copy(
                        rows.at[s], out_hbm.at[pl.ds(r_prev, CHUNK)],
                        osems[s]).wait()

                pltpu.async_copy(tab_hbm.at[idx.at[s]], rows.at[s], gsems[s])
            for s in range(NBUF):
                r0 = base + (p * NBUF + s) * CHUNK
                pltpu.make_async_copy(
                    tab_hbm.at[idx.at[s]], rows.at[s], gsems[s]).wait()
                pltpu.async_copy(
                    rows.at[s], out_hbm.at[pl.ds(r0, CHUNK)], osems[s])
            return carry

        lax.fori_loop(0, n_rounds, round_body, 0)
        for s in range(NBUF):
            r_last = base + ((n_rounds - 1) * NBUF + s) * CHUNK
            pltpu.make_async_copy(
                rows.at[s], out_hbm.at[pl.ds(r_last, CHUNK)], osems[s]).wait()

    return k(X_flat, tables_flat)


def kernel(X, tables):
    F, V, H = tables.shape
    B = X.shape[0]
    X_flat = X.reshape(B * F).astype(jnp.int32)
    tables_flat = tables.reshape(F * V, H)
    out = _embed_gather(X_flat, tables_flat, B=B, F=F, V=V, H=H)
    return out.reshape(B, F * H)
